# baseline (speedup 1.0000x reference)
import jax
import jax.numpy as jnp
from jax.experimental import pallas as pl
from jax.experimental.pallas import tpu as pltpu


def _extract_body(x_hbm_ref, o_hbm_ref, buf, s_in, s_out):
    B = buf.shape[0]
    reads = [
        pltpu.make_async_copy(x_hbm_ref.at[b, 0, :], buf.at[b], s_in)
        for b in range(B)
    ]
    for r in reads:
        r.start()
    writes = []
    for b in range(B):
        reads[b].wait()
        w = pltpu.make_async_copy(buf.at[b], o_hbm_ref.at[b], s_out)
        w.start()
        writes.append(w)
    for w in writes:
        w.wait()


def kernel(inputs):
    B, S, D = inputs.shape
    return pl.pallas_call(
        _extract_body,
        in_specs=[pl.BlockSpec(memory_space=pl.ANY)],
        out_specs=pl.BlockSpec(memory_space=pl.ANY),
        out_shape=jax.ShapeDtypeStruct((B, D), inputs.dtype),
        scratch_shapes=[
            pltpu.VMEM((B, D), inputs.dtype),
            pltpu.SemaphoreType.DMA,
            pltpu.SemaphoreType.DMA,
        ],
    )(inputs)


# speedup vs baseline: 1.0141x; 1.0141x over previous
import jax
import jax.numpy as jnp
from jax.experimental import pallas as pl
from jax.experimental.pallas import tpu as pltpu


def _extract_body(x_hbm_ref, o_ref, sem):
    B = o_ref.shape[0]
    copies = [
        pltpu.make_async_copy(x_hbm_ref.at[b, 0, :], o_ref.at[b], sem)
        for b in range(B)
    ]
    for c in copies:
        c.start()
    for c in copies:
        c.wait()


def kernel(inputs):
    B, S, D = inputs.shape
    return pl.pallas_call(
        _extract_body,
        in_specs=[pl.BlockSpec(memory_space=pl.ANY)],
        out_specs=pl.BlockSpec((B, D), lambda: (0, 0)),
        out_shape=jax.ShapeDtypeStruct((B, D), inputs.dtype),
        scratch_shapes=[pltpu.SemaphoreType.DMA],
    )(inputs)
